# baseline (speedup 1.0000x reference)
import functools

import jax
import jax.numpy as jnp
from jax import lax
from jax.experimental import pallas as pl
from jax.experimental.pallas import tpu as pltpu
from jax.experimental.pallas import tpu_sc as plsc

N = 10000
E = 320000
F = 128
H = 256
O = 128
EPS = 1e-5

NC = 2
NS = 16
NW = NC * NS
CHUNK = 128
N_PAD = 10240
CPT = 2 * (-(-E // (NW * CHUNK * 2)))
EPT = CPT * CHUNK
E_PAD = NW * EPT
PH = CPT // 2
CPT2 = E_PAD // (NS * CHUNK)
N_PHASES = CPT2 // PH

_ROWS_PT = N_PAD // NS


def _sc_mesh():
    return plsc.VectorSubcoreMesh(
        core_axis_name="c", subcore_axis_name="s",
        num_cores=NC, num_subcores=NS)


@functools.cache
def _build_deg_kernel():
    return functools.partial(
        pl.kernel,
        out_type=jax.ShapeDtypeStruct((NC, N_PAD), jnp.float32),
        mesh=_sc_mesh(),
        scratch_types=[
            pltpu.VMEM((CPT, CHUNK), jnp.int32),
            pltpu.VMEM((CHUNK,), jnp.float32),
            pltpu.VMEM_SHARED((N_PAD,), jnp.float32),
            pltpu.SemaphoreType.DMA,
        ],
    )(_deg_body)


def _deg_body(dst_hbm, zeros_hbm, out_hbm, dst_v, ones_v, deg_sh, sem):
    c = lax.axis_index("c")
    s = lax.axis_index("s")
    wid = s * NC + c
    r0 = s * _ROWS_PT
    pltpu.sync_copy(zeros_hbm.at[pl.ds(r0, _ROWS_PT)],
                    deg_sh.at[pl.ds(r0, _ROWS_PT)])
    pltpu.sync_copy(dst_hbm.at[wid], dst_v)
    for i in range(CHUNK // 16):
        ones_v[pl.ds(16 * i, 16)] = jnp.ones((16,), jnp.float32)
    plsc.subcore_barrier()

    @pl.loop(0, CPT)
    def _(j):
        pltpu.sync_copy(ones_v, deg_sh.at[dst_v.at[j]], add=True)

    plsc.subcore_barrier()
    pltpu.sync_copy(deg_sh.at[pl.ds(r0, _ROWS_PT)],
                    out_hbm.at[c, pl.ds(r0, _ROWS_PT)])


@functools.cache
def _build_agg_kernel():
    return functools.partial(
        pl.kernel,
        out_type=jax.ShapeDtypeStruct((N_PAD, F), jnp.float32),
        mesh=_sc_mesh(),
        scratch_types=[
            pltpu.VMEM((PH, CHUNK), jnp.int32),
            pltpu.VMEM((PH, CHUNK), jnp.int32),
            pltpu.VMEM((CHUNK, F), jnp.float32),
            pltpu.VMEM((CHUNK, F), jnp.float32),
            pltpu.VMEM_SHARED((N_PAD, F), jnp.float32),
            pltpu.SemaphoreType.DMA,
            pltpu.SemaphoreType.DMA,
        ],
    )(_agg_body)


def _agg_body(y_hbm, src_hbm, dst_hbm, zeros_hbm, out_hbm,
              src_v, dst_v, buf0, buf1, agg_sh, sem0, sem1):
    c = lax.axis_index("c")
    s = lax.axis_index("s")
    r0 = s * _ROWS_PT

    @pl.when(c == 0)
    def _():
        pltpu.sync_copy(zeros_hbm.at[pl.ds(r0, _ROWS_PT)],
                        agg_sh.at[pl.ds(r0, _ROWS_PT)])
    plsc.subcore_barrier()

    def do_phase(ph):
        pltpu.sync_copy(src_hbm.at[s, pl.ds(ph * PH, PH)], src_v)
        pltpu.sync_copy(dst_hbm.at[s, pl.ds(ph * PH, PH)], dst_v)

        pltpu.async_copy(y_hbm.at[src_v.at[0]], buf0, sem0).wait()

        @pl.loop(0, PH, step=2)
        def _(j):
            nxt1 = pltpu.async_copy(y_hbm.at[src_v.at[j + 1]], buf1, sem1)
            pltpu.sync_copy(buf0, agg_sh.at[dst_v.at[j]], add=True)
            nxt1.wait()

            @pl.when(j + 2 < PH)
            def _():
                pltpu.async_copy(y_hbm.at[src_v.at[j + 2]], buf0, sem0)

            pltpu.sync_copy(buf1, agg_sh.at[dst_v.at[j + 1]], add=True)

            @pl.when(j + 2 < PH)
            def _():
                pltpu.make_async_copy(
                    y_hbm.at[src_v.at[j + 2]], buf0, sem0).wait()

    @pl.when(c == 0)
    def _():
        @pl.loop(0, N_PHASES)
        def _(ph):
            do_phase(ph)

    plsc.subcore_barrier()

    @pl.when(c == 0)
    def _():
        pltpu.sync_copy(agg_sh.at[pl.ds(r0, _ROWS_PT)],
                        out_hbm.at[pl.ds(r0, _ROWS_PT)])


_RB = 512
_GRID = N_PAD // _RB


def _prep_body(p0, p1, x, dinv_ref, y_ref):
    deg = p0[...] + p1[...] + 1.0
    dinv = lax.rsqrt(deg)
    dinv_ref[...] = dinv
    y_ref[...] = dinv * x[...]


def _tc_prep(p0, p1, x):
    return pl.pallas_call(
        _prep_body,
        grid=(_GRID,),
        in_specs=[
            pl.BlockSpec((_RB, 1), lambda i: (i, 0)),
            pl.BlockSpec((_RB, 1), lambda i: (i, 0)),
            pl.BlockSpec((_RB, F), lambda i: (i, 0)),
        ],
        out_specs=[
            pl.BlockSpec((_RB, 1), lambda i: (i, 0)),
            pl.BlockSpec((_RB, F), lambda i: (i, 0)),
        ],
        out_shape=[
            jax.ShapeDtypeStruct((N_PAD, 1), jnp.float32),
            jax.ShapeDtypeStruct((N_PAD, F), jnp.float32),
        ],
    )(p0, p1, x)


def _l1_body(a, y, dinv, w, b, t_ref, stats_ref):
    i = pl.program_id(0)
    pre = dinv[...] * (a[...] + y[...])
    t = jnp.dot(pre, w[...], preferred_element_type=jnp.float32,
                precision=lax.Precision.HIGHEST) + b[...]
    t_ref[...] = t
    rows = i * _RB + lax.broadcasted_iota(jnp.int32, (_RB, 1), 0)
    tm = jnp.where(rows < N, t, 0.0)
    st = jnp.concatenate(
        [jnp.sum(tm, axis=0, keepdims=True),
         jnp.sum(tm * tm, axis=0, keepdims=True)], axis=0)

    @pl.when(i == 0)
    def _():
        stats_ref[...] = st

    @pl.when(i > 0)
    def _():
        stats_ref[...] += st


def _tc_layer1(a, y, dinv, w1, b1):
    return pl.pallas_call(
        _l1_body,
        grid=(_GRID,),
        in_specs=[
            pl.BlockSpec((_RB, F), lambda i: (i, 0)),
            pl.BlockSpec((_RB, F), lambda i: (i, 0)),
            pl.BlockSpec((_RB, 1), lambda i: (i, 0)),
            pl.BlockSpec((F, H), lambda i: (0, 0)),
            pl.BlockSpec((1, H), lambda i: (0, 0)),
        ],
        out_specs=[
            pl.BlockSpec((_RB, H), lambda i: (i, 0)),
            pl.BlockSpec((2, H), lambda i: (0, 0)),
        ],
        out_shape=[
            jax.ShapeDtypeStruct((N_PAD, H), jnp.float32),
            jax.ShapeDtypeStruct((2, H), jnp.float32),
        ],
    )(a, y, dinv, w1, b1)


def _l2_body(t, stats, gamma, beta, dinv, w2, z2_ref):
    mean = stats[0:1, :] * (1.0 / N)
    var = stats[1:2, :] * (1.0 / N) - mean * mean
    scale = lax.rsqrt(var + EPS) * gamma[...]
    h = jnp.maximum((t[...] - mean) * scale + beta[...], 0.0)
    z2_ref[...] = dinv[...] * jnp.dot(
        h, w2[...], preferred_element_type=jnp.float32,
        precision=lax.Precision.HIGHEST)


def _tc_layer2(t, stats, gamma, beta, dinv, w2):
    return pl.pallas_call(
        _l2_body,
        grid=(_GRID,),
        in_specs=[
            pl.BlockSpec((_RB, H), lambda i: (i, 0)),
            pl.BlockSpec((2, H), lambda i: (0, 0)),
            pl.BlockSpec((1, H), lambda i: (0, 0)),
            pl.BlockSpec((1, H), lambda i: (0, 0)),
            pl.BlockSpec((_RB, 1), lambda i: (i, 0)),
            pl.BlockSpec((H, O), lambda i: (0, 0)),
        ],
        out_specs=pl.BlockSpec((_RB, O), lambda i: (i, 0)),
        out_shape=jax.ShapeDtypeStruct((N_PAD, O), jnp.float32),
    )(t, stats, gamma, beta, dinv, w2)


def _fin_body(g, z, dinv, b2, out_ref):
    out_ref[...] = dinv[...] * (g[...] + z[...]) + b2[...]


def _tc_final(g, z, dinv, b2):
    return pl.pallas_call(
        _fin_body,
        grid=(_GRID,),
        in_specs=[
            pl.BlockSpec((_RB, O), lambda i: (i, 0)),
            pl.BlockSpec((_RB, O), lambda i: (i, 0)),
            pl.BlockSpec((_RB, 1), lambda i: (i, 0)),
            pl.BlockSpec((1, O), lambda i: (0, 0)),
        ],
        out_specs=pl.BlockSpec((_RB, O), lambda i: (i, 0)),
        out_shape=jax.ShapeDtypeStruct((N_PAD, O), jnp.float32),
    )(g, z, dinv, b2)


def kernel(x, edge_index, W1, b1, gamma, beta, W2, b2):
    pad = E_PAD - E
    src = jnp.concatenate([edge_index[0], jnp.full((pad,), N, jnp.int32)])
    dst = jnp.concatenate([edge_index[1], jnp.full((pad,), N, jnp.int32)])
    src_slab = src.reshape(NS, CPT2, CHUNK)
    dst_slab = dst.reshape(NS, CPT2, CHUNK)
    dst_slab_deg = dst.reshape(NW, CPT, CHUNK)
    x_pad = jnp.concatenate(
        [x, jnp.zeros((N_PAD - N, F), jnp.float32)], axis=0)
    zeros_vec = jnp.zeros((N_PAD,), jnp.float32)
    zeros_rows = jnp.zeros((N_PAD, F), jnp.float32)

    degp = _build_deg_kernel()(dst_slab_deg, zeros_vec)
    dinv, y1 = _tc_prep(degp[0].reshape(N_PAD, 1),
                        degp[1].reshape(N_PAD, 1), x_pad)

    a1 = _build_agg_kernel()(y1, src_slab, dst_slab, zeros_rows)
    t, stats = _tc_layer1(a1, y1, dinv, W1, b1.reshape(1, H))

    z2 = _tc_layer2(t, stats, gamma.reshape(1, H), beta.reshape(1, H),
                    dinv, W2)
    g = _build_agg_kernel()(z2, src_slab, dst_slab, zeros_rows)
    out = _tc_final(g, z2, dinv, b2.reshape(1, O))
    return out[:N]


# speedup vs baseline: 1.0648x; 1.0648x over previous
import functools

import jax
import jax.numpy as jnp
from jax import lax
from jax.experimental import pallas as pl
from jax.experimental.pallas import tpu as pltpu
from jax.experimental.pallas import tpu_sc as plsc

N = 10000
E = 320000
F = 128
H = 256
O = 128
EPS = 1e-5

NC = 2
NS = 16
NW = NC * NS
CHUNK = 128
N_PAD = 10240
CPT = 2 * (-(-E // (NW * CHUNK * 2)))
EPT = CPT * CHUNK
E_PAD = NW * EPT
QH = 20
NQS = E_PAD // (QH * CHUNK)
Q_FAST = 7
Q_SLOW = 8 - Q_FAST

_ROWS_PT = N_PAD // NS


def _sc_mesh():
    return plsc.VectorSubcoreMesh(
        core_axis_name="c", subcore_axis_name="s",
        num_cores=NC, num_subcores=NS)


@functools.cache
def _build_deg_kernel():
    return functools.partial(
        pl.kernel,
        out_type=jax.ShapeDtypeStruct((NC, N_PAD), jnp.float32),
        mesh=_sc_mesh(),
        scratch_types=[
            pltpu.VMEM((CPT, CHUNK), jnp.int32),
            pltpu.VMEM((CHUNK,), jnp.float32),
            pltpu.VMEM_SHARED((N_PAD,), jnp.float32),
            pltpu.SemaphoreType.DMA,
        ],
    )(_deg_body)


def _deg_body(dst_hbm, zeros_hbm, out_hbm, dst_v, ones_v, deg_sh, sem):
    c = lax.axis_index("c")
    s = lax.axis_index("s")
    wid = s * NC + c
    r0 = s * _ROWS_PT
    pltpu.sync_copy(zeros_hbm.at[pl.ds(r0, _ROWS_PT)],
                    deg_sh.at[pl.ds(r0, _ROWS_PT)])
    pltpu.sync_copy(dst_hbm.at[wid], dst_v)
    for i in range(CHUNK // 16):
        ones_v[pl.ds(16 * i, 16)] = jnp.ones((16,), jnp.float32)
    plsc.subcore_barrier()

    @pl.loop(0, CPT)
    def _(j):
        pltpu.sync_copy(ones_v, deg_sh.at[dst_v.at[j]], add=True)

    plsc.subcore_barrier()
    pltpu.sync_copy(deg_sh.at[pl.ds(r0, _ROWS_PT)],
                    out_hbm.at[c, pl.ds(r0, _ROWS_PT)])


@functools.cache
def _build_agg_kernel():
    return functools.partial(
        pl.kernel,
        out_type=jax.ShapeDtypeStruct((NC, N_PAD, F), jnp.float32),
        mesh=_sc_mesh(),
        scratch_types=[
            pltpu.VMEM((QH, CHUNK), jnp.int32),
            pltpu.VMEM((QH, CHUNK), jnp.int32),
            pltpu.VMEM((CHUNK, F), jnp.float32),
            pltpu.VMEM((CHUNK, F), jnp.float32),
            pltpu.VMEM_SHARED((N_PAD, F), jnp.float32),
            pltpu.SemaphoreType.DMA,
            pltpu.SemaphoreType.DMA,
        ],
    )(_agg_body)


def _agg_body(y_hbm, src_hbm, dst_hbm, zeros_hbm, out_hbm,
              src_v, dst_v, buf0, buf1, agg_sh, sem0, sem1):
    c = lax.axis_index("c")
    s = lax.axis_index("s")
    r0 = s * _ROWS_PT
    pltpu.sync_copy(zeros_hbm.at[pl.ds(r0, _ROWS_PT)],
                    agg_sh.at[pl.ds(r0, _ROWS_PT)])
    plsc.subcore_barrier()

    def do_unit(q):
        pltpu.sync_copy(src_hbm.at[q], src_v)
        pltpu.sync_copy(dst_hbm.at[q], dst_v)

        pltpu.async_copy(y_hbm.at[src_v.at[0]], buf0, sem0).wait()

        @pl.loop(0, QH, step=2)
        def _(j):
            nxt1 = pltpu.async_copy(y_hbm.at[src_v.at[j + 1]], buf1, sem1)
            pltpu.sync_copy(buf0, agg_sh.at[dst_v.at[j]], add=True)
            nxt1.wait()

            @pl.when(j + 2 < QH)
            def _():
                pltpu.async_copy(y_hbm.at[src_v.at[j + 2]], buf0, sem0)

            pltpu.sync_copy(buf1, agg_sh.at[dst_v.at[j + 1]], add=True)

            @pl.when(j + 2 < QH)
            def _():
                pltpu.make_async_copy(
                    y_hbm.at[src_v.at[j + 2]], buf0, sem0).wait()

    @pl.when(c == 0)
    def _():
        @pl.loop(0, Q_FAST)
        def _(q):
            do_unit(s * (Q_FAST + Q_SLOW) + q)

    @pl.when(c == 1)
    def _():
        @pl.loop(0, Q_SLOW)
        def _(q):
            do_unit(s * (Q_FAST + Q_SLOW) + Q_FAST + q)

    plsc.subcore_barrier()
    pltpu.sync_copy(agg_sh.at[pl.ds(r0, _ROWS_PT)],
                    out_hbm.at[c, pl.ds(r0, _ROWS_PT)])


_RB = 512
_GRID = N_PAD // _RB


def _prep_body(p0, p1, x, dinv_ref, y_ref):
    deg = p0[...] + p1[...] + 1.0
    dinv = lax.rsqrt(deg)
    dinv_ref[...] = dinv
    y_ref[...] = dinv * x[...]


def _tc_prep(p0, p1, x):
    return pl.pallas_call(
        _prep_body,
        grid=(_GRID,),
        in_specs=[
            pl.BlockSpec((_RB, 1), lambda i: (i, 0)),
            pl.BlockSpec((_RB, 1), lambda i: (i, 0)),
            pl.BlockSpec((_RB, F), lambda i: (i, 0)),
        ],
        out_specs=[
            pl.BlockSpec((_RB, 1), lambda i: (i, 0)),
            pl.BlockSpec((_RB, F), lambda i: (i, 0)),
        ],
        out_shape=[
            jax.ShapeDtypeStruct((N_PAD, 1), jnp.float32),
            jax.ShapeDtypeStruct((N_PAD, F), jnp.float32),
        ],
    )(p0, p1, x)


def _l1_body(a0, a1, y, dinv, w, b, t_ref, stats_ref):
    i = pl.program_id(0)
    pre = dinv[...] * (a0[...] + a1[...] + y[...])
    t = jnp.dot(pre, w[...], preferred_element_type=jnp.float32,
                precision=lax.Precision.HIGHEST) + b[...]
    t_ref[...] = t
    rows = i * _RB + lax.broadcasted_iota(jnp.int32, (_RB, 1), 0)
    tm = jnp.where(rows < N, t, 0.0)
    st = jnp.concatenate(
        [jnp.sum(tm, axis=0, keepdims=True),
         jnp.sum(tm * tm, axis=0, keepdims=True)], axis=0)

    @pl.when(i == 0)
    def _():
        stats_ref[...] = st

    @pl.when(i > 0)
    def _():
        stats_ref[...] += st


def _tc_layer1(a0, a1, y, dinv, w1, b1):
    return pl.pallas_call(
        _l1_body,
        grid=(_GRID,),
        in_specs=[
            pl.BlockSpec((_RB, F), lambda i: (i, 0)),
            pl.BlockSpec((_RB, F), lambda i: (i, 0)),
            pl.BlockSpec((_RB, F), lambda i: (i, 0)),
            pl.BlockSpec((_RB, 1), lambda i: (i, 0)),
            pl.BlockSpec((F, H), lambda i: (0, 0)),
            pl.BlockSpec((1, H), lambda i: (0, 0)),
        ],
        out_specs=[
            pl.BlockSpec((_RB, H), lambda i: (i, 0)),
            pl.BlockSpec((2, H), lambda i: (0, 0)),
        ],
        out_shape=[
            jax.ShapeDtypeStruct((N_PAD, H), jnp.float32),
            jax.ShapeDtypeStruct((2, H), jnp.float32),
        ],
    )(a0, a1, y, dinv, w1, b1)


def _l2_body(t, stats, gamma, beta, dinv, w2, z_ref):
    mean = stats[0:1, :] * (1.0 / N)
    var = stats[1:2, :] * (1.0 / N) - mean * mean
    scale = lax.rsqrt(var + EPS) * gamma[...]
    h = jnp.maximum((t[...] - mean) * scale + beta[...], 0.0)
    z_ref[...] = dinv[...] * jnp.dot(
        h, w2[...], preferred_element_type=jnp.float32,
        precision=lax.Precision.HIGHEST)


def _tc_layer2(t, stats, gamma, beta, dinv, w2):
    return pl.pallas_call(
        _l2_body,
        grid=(_GRID,),
        in_specs=[
            pl.BlockSpec((_RB, H), lambda i: (i, 0)),
            pl.BlockSpec((2, H), lambda i: (0, 0)),
            pl.BlockSpec((1, H), lambda i: (0, 0)),
            pl.BlockSpec((1, H), lambda i: (0, 0)),
            pl.BlockSpec((_RB, 1), lambda i: (i, 0)),
            pl.BlockSpec((H, O), lambda i: (0, 0)),
        ],
        out_specs=pl.BlockSpec((_RB, O), lambda i: (i, 0)),
        out_shape=jax.ShapeDtypeStruct((N_PAD, O), jnp.float32),
    )(t, stats, gamma, beta, dinv, w2)


def _fin_body(g0, g1, z, dinv, b2, out_ref):
    out_ref[...] = dinv[...] * (g0[...] + g1[...] + z[...]) + b2[...]


def _tc_final(g0, g1, z, dinv, b2):
    return pl.pallas_call(
        _fin_body,
        grid=(_GRID,),
        in_specs=[
            pl.BlockSpec((_RB, O), lambda i: (i, 0)),
            pl.BlockSpec((_RB, O), lambda i: (i, 0)),
            pl.BlockSpec((_RB, O), lambda i: (i, 0)),
            pl.BlockSpec((_RB, 1), lambda i: (i, 0)),
            pl.BlockSpec((1, O), lambda i: (0, 0)),
        ],
        out_specs=pl.BlockSpec((_RB, O), lambda i: (i, 0)),
        out_shape=jax.ShapeDtypeStruct((N_PAD, O), jnp.float32),
    )(g0, g1, z, dinv, b2)


def kernel(x, edge_index, W1, b1, gamma, beta, W2, b2):
    pad = E_PAD - E
    src = jnp.concatenate([edge_index[0], jnp.full((pad,), N, jnp.int32)])
    dst = jnp.concatenate([edge_index[1], jnp.full((pad,), N, jnp.int32)])
    src_slab = src.reshape(NQS, QH, CHUNK)
    dst_slab = dst.reshape(NQS, QH, CHUNK)
    dst_slab_deg = dst.reshape(NW, CPT, CHUNK)
    x_pad = jnp.concatenate(
        [x, jnp.zeros((N_PAD - N, F), jnp.float32)], axis=0)
    zeros_vec = jnp.zeros((N_PAD,), jnp.float32)
    zeros_rows = jnp.zeros((N_PAD, F), jnp.float32)

    degp = _build_deg_kernel()(dst_slab_deg, zeros_vec)
    dinv, y1 = _tc_prep(degp[0].reshape(N_PAD, 1),
                        degp[1].reshape(N_PAD, 1), x_pad)

    agg1 = _build_agg_kernel()(y1, src_slab, dst_slab, zeros_rows)
    t, stats = _tc_layer1(agg1[0], agg1[1], y1, dinv,
                          W1, b1.reshape(1, H))

    z2 = _tc_layer2(t, stats, gamma.reshape(1, H), beta.reshape(1, H),
                    dinv, W2)
    agg2 = _build_agg_kernel()(z2, src_slab, dst_slab, zeros_rows)
    out = _tc_final(agg2[0], agg2[1], z2, dinv, b2.reshape(1, O))
    return out[:N]


# speedup vs baseline: 1.2457x; 1.1699x over previous
import functools

import jax
import jax.numpy as jnp
from jax import lax
from jax.experimental import pallas as pl
from jax.experimental.pallas import tpu as pltpu
from jax.experimental.pallas import tpu_sc as plsc

N = 10000
E = 320000
F = 128
H = 256
O = 128
EPS = 1e-5

NC = 2
NS = 16
NW = NC * NS
CHUNK = 128
N_PAD = 10240
CPT = 2 * (-(-E // (NW * CHUNK * 2)))
EPT = CPT * CHUNK
E_PAD = NW * EPT
QH = 40
NQS = E_PAD // (QH * CHUNK)
Q_FAST = 3
Q_SLOW = 4 - Q_FAST

_ROWS_PT = N_PAD // NS


def _sc_mesh():
    return plsc.VectorSubcoreMesh(
        core_axis_name="c", subcore_axis_name="s",
        num_cores=NC, num_subcores=NS)


@functools.cache
def _build_deg_kernel():
    return functools.partial(
        pl.kernel,
        out_type=jax.ShapeDtypeStruct((NC, N_PAD), jnp.float32),
        mesh=_sc_mesh(),
        scratch_types=[
            pltpu.VMEM((CPT, CHUNK), jnp.int32),
            pltpu.VMEM((CHUNK,), jnp.float32),
            pltpu.VMEM_SHARED((N_PAD,), jnp.float32),
            pltpu.SemaphoreType.DMA,
        ],
    )(_deg_body)


def _deg_body(dst_hbm, zeros_hbm, out_hbm, dst_v, ones_v, deg_sh, sem):
    c = lax.axis_index("c")
    s = lax.axis_index("s")
    wid = s * NC + c
    r0 = s * _ROWS_PT
    pltpu.sync_copy(zeros_hbm.at[pl.ds(r0, _ROWS_PT)],
                    deg_sh.at[pl.ds(r0, _ROWS_PT)])
    pltpu.sync_copy(dst_hbm.at[wid], dst_v)
    for i in range(CHUNK // 16):
        ones_v[pl.ds(16 * i, 16)] = jnp.ones((16,), jnp.float32)
    plsc.subcore_barrier()

    @pl.loop(0, CPT)
    def _(j):
        pltpu.sync_copy(ones_v, deg_sh.at[dst_v.at[j]], add=True)

    plsc.subcore_barrier()
    pltpu.sync_copy(deg_sh.at[pl.ds(r0, _ROWS_PT)],
                    out_hbm.at[c, pl.ds(r0, _ROWS_PT)])


@functools.cache
def _build_agg_kernel():
    return functools.partial(
        pl.kernel,
        out_type=jax.ShapeDtypeStruct((NC, N_PAD, F), jnp.float32),
        mesh=_sc_mesh(),
        scratch_types=[
            pltpu.VMEM((QH, CHUNK), jnp.int32),
            pltpu.VMEM((QH, CHUNK), jnp.int32),
            pltpu.VMEM((CHUNK, F), jnp.float32),
            pltpu.VMEM((CHUNK, F), jnp.float32),
            pltpu.VMEM_SHARED((N_PAD, F), jnp.float32),
            pltpu.SemaphoreType.DMA,
            pltpu.SemaphoreType.DMA,
        ],
    )(_agg_body)


def _agg_body(y_hbm, src_hbm, dst_hbm, zeros_hbm, out_hbm,
              src_v, dst_v, buf0, buf1, agg_sh, sem0, sem1):
    c = lax.axis_index("c")
    s = lax.axis_index("s")
    r0 = s * _ROWS_PT
    pltpu.sync_copy(zeros_hbm.at[pl.ds(r0, _ROWS_PT)],
                    agg_sh.at[pl.ds(r0, _ROWS_PT)])
    plsc.subcore_barrier()

    def do_unit(q):
        pltpu.sync_copy(src_hbm.at[q], src_v)
        pltpu.sync_copy(dst_hbm.at[q], dst_v)

        pltpu.async_copy(y_hbm.at[src_v.at[0]], buf0, sem0).wait()

        @pl.loop(0, QH, step=2)
        def _(j):
            nxt1 = pltpu.async_copy(y_hbm.at[src_v.at[j + 1]], buf1, sem1)
            pltpu.sync_copy(buf0, agg_sh.at[dst_v.at[j]], add=True)
            nxt1.wait()

            @pl.when(j + 2 < QH)
            def _():
                pltpu.async_copy(y_hbm.at[src_v.at[j + 2]], buf0, sem0)

            pltpu.sync_copy(buf1, agg_sh.at[dst_v.at[j + 1]], add=True)

            @pl.when(j + 2 < QH)
            def _():
                pltpu.make_async_copy(
                    y_hbm.at[src_v.at[j + 2]], buf0, sem0).wait()

    @pl.when(c == 0)
    def _():
        @pl.loop(0, Q_FAST)
        def _(q):
            do_unit(s * (Q_FAST + Q_SLOW) + q)

    @pl.when(c == 1)
    def _():
        @pl.loop(0, Q_SLOW)
        def _(q):
            do_unit(s * (Q_FAST + Q_SLOW) + Q_FAST + q)

    plsc.subcore_barrier()
    pltpu.sync_copy(agg_sh.at[pl.ds(r0, _ROWS_PT)],
                    out_hbm.at[c, pl.ds(r0, _ROWS_PT)])


_RB = 512
_GRID = N_PAD // _RB


def _prep_body(p0, p1, x, dinv_ref, y_ref):
    deg = p0[...] + p1[...] + 1.0
    dinv = lax.rsqrt(deg)
    dinv_ref[...] = dinv
    y_ref[...] = dinv * x[...]


def _tc_prep(p0, p1, x):
    return pl.pallas_call(
        _prep_body,
        grid=(_GRID,),
        in_specs=[
            pl.BlockSpec((_RB, 1), lambda i: (i, 0)),
            pl.BlockSpec((_RB, 1), lambda i: (i, 0)),
            pl.BlockSpec((_RB, F), lambda i: (i, 0)),
        ],
        out_specs=[
            pl.BlockSpec((_RB, 1), lambda i: (i, 0)),
            pl.BlockSpec((_RB, F), lambda i: (i, 0)),
        ],
        out_shape=[
            jax.ShapeDtypeStruct((N_PAD, 1), jnp.float32),
            jax.ShapeDtypeStruct((N_PAD, F), jnp.float32),
        ],
    )(p0, p1, x)


def _l1_body(a0, a1, y, dinv, w, b, t_ref, stats_ref):
    i = pl.program_id(0)
    pre = dinv[...] * (a0[...] + a1[...] + y[...])
    t = jnp.dot(pre, w[...], preferred_element_type=jnp.float32,
                precision=lax.Precision.HIGHEST) + b[...]
    t_ref[...] = t
    rows = i * _RB + lax.broadcasted_iota(jnp.int32, (_RB, 1), 0)
    tm = jnp.where(rows < N, t, 0.0)
    st = jnp.concatenate(
        [jnp.sum(tm, axis=0, keepdims=True),
         jnp.sum(tm * tm, axis=0, keepdims=True)], axis=0)

    @pl.when(i == 0)
    def _():
        stats_ref[...] = st

    @pl.when(i > 0)
    def _():
        stats_ref[...] += st


def _tc_layer1(a0, a1, y, dinv, w1, b1):
    return pl.pallas_call(
        _l1_body,
        grid=(_GRID,),
        in_specs=[
            pl.BlockSpec((_RB, F), lambda i: (i, 0)),
            pl.BlockSpec((_RB, F), lambda i: (i, 0)),
            pl.BlockSpec((_RB, F), lambda i: (i, 0)),
            pl.BlockSpec((_RB, 1), lambda i: (i, 0)),
            pl.BlockSpec((F, H), lambda i: (0, 0)),
            pl.BlockSpec((1, H), lambda i: (0, 0)),
        ],
        out_specs=[
            pl.BlockSpec((_RB, H), lambda i: (i, 0)),
            pl.BlockSpec((2, H), lambda i: (0, 0)),
        ],
        out_shape=[
            jax.ShapeDtypeStruct((N_PAD, H), jnp.float32),
            jax.ShapeDtypeStruct((2, H), jnp.float32),
        ],
    )(a0, a1, y, dinv, w1, b1)


def _l2_body(t, stats, gamma, beta, dinv, w2, z_ref):
    mean = stats[0:1, :] * (1.0 / N)
    var = stats[1:2, :] * (1.0 / N) - mean * mean
    scale = lax.rsqrt(var + EPS) * gamma[...]
    h = jnp.maximum((t[...] - mean) * scale + beta[...], 0.0)
    z_ref[...] = dinv[...] * jnp.dot(
        h, w2[...], preferred_element_type=jnp.float32,
        precision=lax.Precision.HIGHEST)


def _tc_layer2(t, stats, gamma, beta, dinv, w2):
    return pl.pallas_call(
        _l2_body,
        grid=(_GRID,),
        in_specs=[
            pl.BlockSpec((_RB, H), lambda i: (i, 0)),
            pl.BlockSpec((2, H), lambda i: (0, 0)),
            pl.BlockSpec((1, H), lambda i: (0, 0)),
            pl.BlockSpec((1, H), lambda i: (0, 0)),
            pl.BlockSpec((_RB, 1), lambda i: (i, 0)),
            pl.BlockSpec((H, O), lambda i: (0, 0)),
        ],
        out_specs=pl.BlockSpec((_RB, O), lambda i: (i, 0)),
        out_shape=jax.ShapeDtypeStruct((N_PAD, O), jnp.float32),
    )(t, stats, gamma, beta, dinv, w2)


def _fin_body(g0, g1, z, dinv, b2, out_ref):
    out_ref[...] = dinv[...] * (g0[...] + g1[...] + z[...]) + b2[...]


def _tc_final(g0, g1, z, dinv, b2):
    return pl.pallas_call(
        _fin_body,
        grid=(_GRID,),
        in_specs=[
            pl.BlockSpec((_RB, O), lambda i: (i, 0)),
            pl.BlockSpec((_RB, O), lambda i: (i, 0)),
            pl.BlockSpec((_RB, O), lambda i: (i, 0)),
            pl.BlockSpec((_RB, 1), lambda i: (i, 0)),
            pl.BlockSpec((1, O), lambda i: (0, 0)),
        ],
        out_specs=pl.BlockSpec((_RB, O), lambda i: (i, 0)),
        out_shape=jax.ShapeDtypeStruct((N_PAD, O), jnp.float32),
    )(g0, g1, z, dinv, b2)


def kernel(x, edge_index, W1, b1, gamma, beta, W2, b2):
    pad = E_PAD - E
    src = jnp.concatenate([edge_index[0], jnp.full((pad,), N, jnp.int32)])
    dst = jnp.concatenate([edge_index[1], jnp.full((pad,), N, jnp.int32)])
    src_slab = src.reshape(NQS, QH, CHUNK)
    dst_slab = dst.reshape(NQS, QH, CHUNK)
    dst_slab_deg = dst.reshape(NW, CPT, CHUNK)
    x_pad = jnp.concatenate(
        [x, jnp.zeros((N_PAD - N, F), jnp.float32)], axis=0)
    zeros_vec = jnp.zeros((N_PAD,), jnp.float32)
    zeros_rows = jnp.zeros((N_PAD, F), jnp.float32)

    degp = _build_deg_kernel()(dst_slab_deg, zeros_vec)
    dinv, y1 = _tc_prep(degp[0].reshape(N_PAD, 1),
                        degp[1].reshape(N_PAD, 1), x_pad)

    agg1 = _build_agg_kernel()(y1, src_slab, dst_slab, zeros_rows)
    t, stats = _tc_layer1(agg1[0], agg1[1], y1, dinv,
                          W1, b1.reshape(1, H))

    z2 = _tc_layer2(t, stats, gamma.reshape(1, H), beta.reshape(1, H),
                    dinv, W2)
    agg2 = _build_agg_kernel()(z2, src_slab, dst_slab, zeros_rows)
    out = _tc_final(agg2[0], agg2[1], z2, dinv, b2.reshape(1, O))
    return out[:N]
